# baseline (speedup 1.0000x reference)
import functools
import jax
import jax.numpy as jnp
from jax import lax
from jax.experimental import pallas as pl
from jax.experimental.pallas import tpu as pltpu
from jax.experimental.pallas import tpu_sc as plsc

B = 16384
D = 128
NEG = 10
K = 64
GAMMA = 0.01

NC = 2
NS = 16
NW = NC * NS
BW = B // NW

CH = 32
NCHUNK = BW // CH
NROWS = CH * NEG
UCH = 128


def _sc_u_body(uidx_hbm, vidx_hbm, emb_u_hbm, emb_v_hbm, u_out, pparts_out,
               uidx_v, vidx_v, urows0, urows1, vrows0, vrows1,
               pparts0, pparts1, sem0, sem1):
    wid = lax.axis_index("s") * NC + lax.axis_index("c")
    base = wid * BW
    urows = (urows0, urows1)
    vrows = (vrows0, vrows1)
    pparts = (pparts0, pparts1)
    sems = (sem0, sem1)

    pltpu.sync_copy(uidx_hbm.at[pl.ds(base, BW)], uidx_v)
    pltpu.sync_copy(vidx_hbm.at[pl.ds(base, BW)], vidx_v)

    def fire(c, slot):
        return (
            pltpu.async_copy(emb_u_hbm.at[uidx_v.at[pl.ds(c * UCH, UCH)]],
                             urows[slot], sems[slot]),
            pltpu.async_copy(emb_v_hbm.at[vidx_v.at[pl.ds(c * UCH, UCH)]],
                             vrows[slot], sems[slot]),
        )

    inflight = fire(0, 0)
    for c in range(BW // UCH):
        slot = c % 2
        cur = inflight
        if c + 1 < BW // UCH:
            inflight = fire(c + 1, 1 - slot)
        for cp in cur:
            cp.wait()
        ur = urows[slot]
        vr = vrows[slot]
        pp = pparts[slot]

        def item_body(i, carry, ur=ur, vr=vr, pp=pp):
            pa = None
            for d in range(D // 16):
                sl = pl.ds(d * 16, 16)
                pd = ur[i, sl] * vr[i, sl]
                pa = pd if pa is None else pa + pd
            pp[i, :] = pa
            return carry

        lax.fori_loop(0, UCH, item_body, 0)
        pltpu.sync_copy(ur, u_out.at[pl.ds(base + c * UCH, UCH)])
        pltpu.sync_copy(pp, pparts_out.at[pl.ds(base + c * UCH, UCH)])


@functools.lru_cache(maxsize=None)
def _make_sc_u():
    return pl.kernel(
        _sc_u_body,
        out_type=(
            jax.ShapeDtypeStruct((B, D), jnp.float32),
            jax.ShapeDtypeStruct((B, 16), jnp.float32),
        ),
        mesh=plsc.VectorSubcoreMesh(core_axis_name="c", subcore_axis_name="s",
                                    num_cores=NC, num_subcores=NS),
        scratch_types=[
            pltpu.VMEM((BW,), jnp.int32),
            pltpu.VMEM((BW,), jnp.int32),
            pltpu.VMEM((UCH, D), jnp.float32),
            pltpu.VMEM((UCH, D), jnp.float32),
            pltpu.VMEM((UCH, D), jnp.float32),
            pltpu.VMEM((UCH, D), jnp.float32),
            pltpu.VMEM((UCH, 16), jnp.float32),
            pltpu.VMEM((UCH, 16), jnp.float32),
            pltpu.SemaphoreType.DMA,
            pltpu.SemaphoreType.DMA,
        ],
    )


def _fire(c, slot, u_emb_hbm, nidx_v, emb_v_hbm, base, urow, nrows, sems):
    cps = [
        pltpu.async_copy(u_emb_hbm.at[pl.ds(base + c * CH, CH)],
                         urow[slot], sems[slot]),
    ]
    for s, ln in ((0, 128), (1, 128), (2, 64)):
        cps.append(pltpu.async_copy(
            emb_v_hbm.at[nidx_v.at[pl.ds(c * NROWS + s * 128, ln)]],
            nrows[slot].at[pl.ds(s * 128, ln)], sems[slot]))
    return cps


def _sc_body(nidx_hbm, u_emb_hbm, emb_v_hbm,
             parts_out,
             nidx_v, urow0, urow1,
             nrows0, nrows1, parts_v0, parts_v1, sem0, sem1, wsem0, wsem1):
    wid = lax.axis_index("s") * NC + lax.axis_index("c")
    base = wid * BW
    urow = (urow0, urow1)
    nrows = (nrows0, nrows1)
    parts = (parts_v0, parts_v1)
    sems = (sem0, sem1)
    wsems = (wsem0, wsem1)

    pltpu.sync_copy(nidx_hbm.at[pl.ds(base * NEG, BW * NEG)], nidx_v)

    inflight = _fire(0, 0, u_emb_hbm, nidx_v, emb_v_hbm,
                     base, urow, nrows, sems)
    pending = [(), ()]

    for c in range(NCHUNK):
        slot = c % 2
        nxt = inflight
        if c + 1 < NCHUNK:
            for wp in pending[1 - slot]:
                wp.wait()
            pending[1 - slot] = ()
            inflight = _fire(c + 1, 1 - slot, u_emb_hbm, nidx_v,
                             emb_v_hbm, base, urow, nrows, sems)
        for cp in nxt:
            cp.wait()

        ur = urow[slot]
        nr = nrows[slot]
        parts_v = parts[slot]

        def item_body(i, carry, ur=ur, nr=nr, parts_v=parts_v):
            r0 = i * NEG
            na = None
            for d in range(D // 16):
                sl = pl.ds(d * 16, 16)
                nsd = nr[r0, sl]
                for j in range(1, NEG):
                    nsd = nsd + nr[r0 + j, sl]
                nd = ur[i, sl] * nsd
                na = nd if na is None else na + nd
            parts_v[i, :] = -na
            return carry

        lax.fori_loop(0, CH, item_body, 0)

        for wp in pending[slot]:
            wp.wait()
        pending[slot] = (
            pltpu.async_copy(parts_v,
                             parts_out.at[pl.ds(base + c * CH, CH)],
                             wsems[slot]),
        )

    for ps in pending:
        for wp in ps:
            wp.wait()


@functools.lru_cache(maxsize=None)
def _make_sc_main():
    return pl.kernel(
        _sc_body,
        out_type=jax.ShapeDtypeStruct((B, 16), jnp.float32),
        mesh=plsc.VectorSubcoreMesh(core_axis_name="c", subcore_axis_name="s",
                                    num_cores=NC, num_subcores=NS),
        scratch_types=[
            pltpu.VMEM((BW * NEG,), jnp.int32),
            pltpu.VMEM((CH, D), jnp.float32),
            pltpu.VMEM((CH, D), jnp.float32),
            pltpu.VMEM((NROWS, D), jnp.float32),
            pltpu.VMEM((NROWS, D), jnp.float32),
            pltpu.VMEM((CH, 16), jnp.float32),
            pltpu.VMEM((CH, 16), jnp.float32),
            pltpu.SemaphoreType.DMA,
            pltpu.SemaphoreType.DMA,
            pltpu.SemaphoreType.DMA,
            pltpu.SemaphoreType.DMA,
        ],
    )


RB = 4096
NGRID = B // RB


def _tc_dist_body(u_ref, com_ref, cluster_ref, ksum_ref, acc_ref):
    step = pl.program_id(0)

    @pl.when(step == 0)
    def _():
        acc_ref[0] = 0.0

    u = u_ref[...]
    com = com_ref[...]

    dots = lax.dot_general(com, u, (((1,), (1,)), ((), ())),
                           preferred_element_type=jnp.float32,
                           precision=lax.Precision.HIGHEST)
    cn2 = jnp.sum(com * com, axis=1)
    m = cn2[:, None] - 2.0 * dots
    ii = lax.broadcasted_iota(jnp.int32, (K, RB), 0)
    for half in (32, 16, 8, 4, 2, 1):
        tm, bm = m[:half], m[half:]
        ti, bi = ii[:half], ii[half:]
        tk = bm < tm
        m = jnp.where(tk, bm, tm)
        ii = jnp.where(tk, bi, ti)
    cluster_ref[...] = ii[0]

    acc_ref[0] += jnp.sum(m) + jnp.sum(u * u)

    @pl.when(step == NGRID - 1)
    def _():
        ksum_ref[0, 0] = acc_ref[0]


_tc_dist = pl.pallas_call(
    _tc_dist_body,
    grid=(NGRID,),
    in_specs=[
        pl.BlockSpec((RB, D), lambda i: (i, 0)),
        pl.BlockSpec((K, D), lambda i: (0, 0)),
    ],
    out_specs=[
        pl.BlockSpec((RB,), lambda i: (i,)),
        pl.BlockSpec(memory_space=pltpu.SMEM, block_shape=(1, 1),
                     index_map=lambda i: (0, 0)),
    ],
    out_shape=[
        jax.ShapeDtypeStruct((B,), jnp.int32),
        jax.ShapeDtypeStruct((1, 1), jnp.float32),
    ],
    scratch_shapes=[pltpu.SMEM((1,), jnp.float32)],
)


def _tc_loss_body(pparts_ref, nparts_ref, ksum_ref, loss_ref):
    ones = jnp.ones((2, 16), jnp.float32)
    pos = lax.dot_general(ones, pparts_ref[...], (((1,), (1,)), ((), ())),
                          preferred_element_type=jnp.float32)
    neg = lax.dot_general(ones, nparts_ref[...], (((1,), (1,)), ((), ())),
                          preferred_element_type=jnp.float32)
    ls_sum = 0.5 * (jnp.sum(jax.nn.log_sigmoid(pos))
                    + jnp.sum(jax.nn.log_sigmoid(neg)))
    loss_ref[0, 0] = -(ls_sum / B) + GAMMA * (ksum_ref[0, 0] / B)


_tc_loss = pl.pallas_call(
    _tc_loss_body,
    in_specs=[
        pl.BlockSpec((B, 16), lambda: (0, 0)),
        pl.BlockSpec((B, 16), lambda: (0, 0)),
        pl.BlockSpec(memory_space=pltpu.SMEM, block_shape=(1, 1),
                     index_map=lambda: (0, 0)),
    ],
    out_specs=pl.BlockSpec(memory_space=pltpu.SMEM, block_shape=(1, 1),
                           index_map=lambda: (0, 0)),
    out_shape=jax.ShapeDtypeStruct((1, 1), jnp.float32),
)


def kernel(u_node, v_node, negative_nodes, emb_u, emb_v, emb_com):
    u_idx = u_node.reshape(B).astype(jnp.int32)
    v_idx = v_node.reshape(B).astype(jnp.int32)
    n_idx = negative_nodes.reshape(B * NEG).astype(jnp.int32)

    u_emb, pparts = _make_sc_u()(u_idx, v_idx, emb_u, emb_v)
    nparts = _make_sc_main()(n_idx, u_emb, emb_v)
    cluster, ksum = _tc_dist(u_emb, emb_com)
    loss = _tc_loss(pparts, nparts, ksum)
    return (jnp.float32(GAMMA), loss.reshape(()), cluster)


# speedup vs baseline: 1.1264x; 1.1264x over previous
import functools
import jax
import jax.numpy as jnp
from jax import lax
from jax.experimental import pallas as pl
from jax.experimental.pallas import tpu as pltpu
from jax.experimental.pallas import tpu_sc as plsc

B = 16384
D = 128
NEG = 10
K = 64
GAMMA = 0.01

NC = 2
NS = 16
NW = NC * NS
BW = B // NW

CH = 32
NCHUNK = BW // CH
NROWS = CH * NEG
UCH = 128


def _sc_u_body(uidx_hbm, emb_u_hbm, u_out, uidx_v, rows0, rows1, sem0, sem1):
    wid = lax.axis_index("s") * NC + lax.axis_index("c")
    base = wid * BW
    rows = (rows0, rows1)
    sems = (sem0, sem1)

    pltpu.sync_copy(uidx_hbm.at[pl.ds(base, BW)], uidx_v)

    def fire(c, slot):
        return pltpu.async_copy(
            emb_u_hbm.at[uidx_v.at[pl.ds(c * UCH, UCH)]], rows[slot],
            sems[slot])

    inflight = fire(0, 0)
    for c in range(BW // UCH):
        slot = c % 2
        cur = inflight
        if c + 1 < BW // UCH:
            inflight = fire(c + 1, 1 - slot)
        cur.wait()
        pltpu.sync_copy(rows[slot], u_out.at[pl.ds(base + c * UCH, UCH)])


@functools.lru_cache(maxsize=None)
def _make_sc_u():
    return pl.kernel(
        _sc_u_body,
        out_type=jax.ShapeDtypeStruct((B, D), jnp.float32),
        mesh=plsc.VectorSubcoreMesh(core_axis_name="c", subcore_axis_name="s",
                                    num_cores=NC, num_subcores=NS),
        scratch_types=[
            pltpu.VMEM((BW,), jnp.int32),
            pltpu.VMEM((UCH, D), jnp.float32),
            pltpu.VMEM((UCH, D), jnp.float32),
            pltpu.SemaphoreType.DMA,
            pltpu.SemaphoreType.DMA,
        ],
    )


def _fire(c, slot, u_emb_hbm, vidx_v, nidx_v, emb_v_hbm,
          base, urow, vrow, nrows, sems):
    cps = [
        pltpu.async_copy(u_emb_hbm.at[pl.ds(base + c * CH, CH)],
                         urow[slot], sems[slot]),
        pltpu.async_copy(emb_v_hbm.at[vidx_v.at[pl.ds(c * CH, CH)]],
                         vrow[slot], sems[slot]),
    ]
    for s, ln in ((0, 128), (1, 128), (2, 64)):
        cps.append(pltpu.async_copy(
            emb_v_hbm.at[nidx_v.at[pl.ds(c * NROWS + s * 128, ln)]],
            nrows[slot].at[pl.ds(s * 128, ln)], sems[slot]))
    return cps


def _sc_body(vidx_hbm, nidx_hbm, u_emb_hbm, emb_v_hbm,
             parts_out,
             vidx_v, nidx_v, urow0, urow1, vrow0, vrow1,
             nrows0, nrows1, parts_v0, parts_v1, sem0, sem1, wsem0, wsem1):
    wid = lax.axis_index("s") * NC + lax.axis_index("c")
    base = wid * BW
    urow = (urow0, urow1)
    vrow = (vrow0, vrow1)
    nrows = (nrows0, nrows1)
    parts = (parts_v0, parts_v1)
    sems = (sem0, sem1)
    wsems = (wsem0, wsem1)

    pltpu.sync_copy(vidx_hbm.at[pl.ds(base, BW)], vidx_v)
    pltpu.sync_copy(nidx_hbm.at[pl.ds(base * NEG, BW * NEG)], nidx_v)

    inflight = _fire(0, 0, u_emb_hbm, vidx_v, nidx_v, emb_v_hbm,
                     base, urow, vrow, nrows, sems)
    pending = [(), ()]

    for c in range(NCHUNK):
        slot = c % 2
        nxt = inflight
        if c + 1 < NCHUNK:
            for wp in pending[1 - slot]:
                wp.wait()
            pending[1 - slot] = ()
            inflight = _fire(c + 1, 1 - slot, u_emb_hbm, vidx_v, nidx_v,
                             emb_v_hbm, base, urow, vrow, nrows, sems)
        for cp in nxt:
            cp.wait()

        ur = urow[slot]
        vr = vrow[slot]
        nr = nrows[slot]
        parts_v = parts[slot]

        def item_body(i, carry, ur=ur, vr=vr, nr=nr, parts_v=parts_v):
            r0 = i * NEG
            pa = None
            na = None
            for d in range(D // 16):
                sl = pl.ds(d * 16, 16)
                ud = ur[i, sl]
                nsd = nr[r0, sl]
                for j in range(1, NEG):
                    nsd = nsd + nr[r0 + j, sl]
                pd = ud * vr[i, sl]
                nd = ud * nsd
                pa = pd if pa is None else pa + pd
                na = nd if na is None else na + nd
            parts_v[i, pl.ds(0, 16)] = pa
            parts_v[i, pl.ds(16, 16)] = -na
            return carry

        lax.fori_loop(0, CH, item_body, 0)

        for wp in pending[slot]:
            wp.wait()
        pending[slot] = (
            pltpu.async_copy(parts_v,
                             parts_out.at[pl.ds(base + c * CH, CH)],
                             wsems[slot]),
        )

    for ps in pending:
        for wp in ps:
            wp.wait()


@functools.lru_cache(maxsize=None)
def _make_sc_main():
    return pl.kernel(
        _sc_body,
        out_type=jax.ShapeDtypeStruct((B, 32), jnp.float32),
        mesh=plsc.VectorSubcoreMesh(core_axis_name="c", subcore_axis_name="s",
                                    num_cores=NC, num_subcores=NS),
        scratch_types=[
            pltpu.VMEM((BW,), jnp.int32),
            pltpu.VMEM((BW * NEG,), jnp.int32),
            pltpu.VMEM((CH, D), jnp.float32),
            pltpu.VMEM((CH, D), jnp.float32),
            pltpu.VMEM((CH, D), jnp.float32),
            pltpu.VMEM((CH, D), jnp.float32),
            pltpu.VMEM((NROWS, D), jnp.float32),
            pltpu.VMEM((NROWS, D), jnp.float32),
            pltpu.VMEM((CH, 32), jnp.float32),
            pltpu.VMEM((CH, 32), jnp.float32),
            pltpu.SemaphoreType.DMA,
            pltpu.SemaphoreType.DMA,
            pltpu.SemaphoreType.DMA,
            pltpu.SemaphoreType.DMA,
        ],
    )


RB = 4096
NGRID = B // RB


def _tc_dist_body(u_ref, com_ref, cluster_ref, ksum_ref, acc_ref):
    step = pl.program_id(0)

    @pl.when(step == 0)
    def _():
        acc_ref[0] = 0.0

    u = u_ref[...]
    com = com_ref[...]

    dots = lax.dot_general(com, u, (((1,), (1,)), ((), ())),
                           preferred_element_type=jnp.float32,
                           precision=lax.Precision.HIGHEST)
    cn2 = jnp.sum(com * com, axis=1)
    m = cn2[:, None] - 2.0 * dots
    ii = lax.broadcasted_iota(jnp.int32, (K, RB), 0)
    for half in (32, 16, 8, 4, 2, 1):
        tm, bm = m[:half], m[half:]
        ti, bi = ii[:half], ii[half:]
        tk = bm < tm
        m = jnp.where(tk, bm, tm)
        ii = jnp.where(tk, bi, ti)
    cluster_ref[...] = ii[0]

    acc_ref[0] += jnp.sum(m) + jnp.sum(u * u)

    @pl.when(step == NGRID - 1)
    def _():
        ksum_ref[0, 0] = acc_ref[0]


_tc_dist = pl.pallas_call(
    _tc_dist_body,
    grid=(NGRID,),
    in_specs=[
        pl.BlockSpec((RB, D), lambda i: (i, 0)),
        pl.BlockSpec((K, D), lambda i: (0, 0)),
    ],
    out_specs=[
        pl.BlockSpec((RB,), lambda i: (i,)),
        pl.BlockSpec(memory_space=pltpu.SMEM, block_shape=(1, 1),
                     index_map=lambda i: (0, 0)),
    ],
    out_shape=[
        jax.ShapeDtypeStruct((B,), jnp.int32),
        jax.ShapeDtypeStruct((1, 1), jnp.float32),
    ],
    scratch_shapes=[pltpu.SMEM((1,), jnp.float32)],
)


def _tc_loss_body(parts_ref, ksum_ref, loss_ref):
    gsel = (lax.broadcasted_iota(jnp.int32, (2, 32), 0)
            == lax.broadcasted_iota(jnp.int32, (2, 32), 1) // 16)
    scores = lax.dot_general(gsel.astype(jnp.float32), parts_ref[...],
                             (((1,), (1,)), ((), ())),
                             preferred_element_type=jnp.float32)
    ls_sum = jnp.sum(jax.nn.log_sigmoid(scores))
    loss_ref[0, 0] = -(ls_sum / B) + GAMMA * (ksum_ref[0, 0] / B)


_tc_loss = pl.pallas_call(
    _tc_loss_body,
    in_specs=[
        pl.BlockSpec((B, 32), lambda: (0, 0)),
        pl.BlockSpec(memory_space=pltpu.SMEM, block_shape=(1, 1),
                     index_map=lambda: (0, 0)),
    ],
    out_specs=pl.BlockSpec(memory_space=pltpu.SMEM, block_shape=(1, 1),
                           index_map=lambda: (0, 0)),
    out_shape=jax.ShapeDtypeStruct((1, 1), jnp.float32),
)


def kernel(u_node, v_node, negative_nodes, emb_u, emb_v, emb_com):
    u_idx = u_node.reshape(B).astype(jnp.int32)
    v_idx = v_node.reshape(B).astype(jnp.int32)
    n_idx = negative_nodes.reshape(B * NEG).astype(jnp.int32)

    u_emb = _make_sc_u()(u_idx, emb_u)
    parts = _make_sc_main()(v_idx, n_idx, u_emb, emb_v)
    cluster, ksum = _tc_dist(u_emb, emb_com)
    loss = _tc_loss(parts, ksum)
    return (jnp.float32(GAMMA), loss.reshape(()), cluster)
